# baseline (speedup 1.0000x reference)
import functools

import jax
import jax.numpy as jnp
from jax import lax
from jax.experimental import pallas as pl
from jax.experimental.pallas import tpu as pltpu
from jax.experimental.pallas import tpu_sc as plsc

N_NODES = 10000
N_EDGES = 320000
D = 128
N_HEADS = 8
D_KV = 16
EPS = 1e-6
GATE_BIAS = 3.0

NC = 2
NS = 16
NW = NC * NS

CHUNK = 128
N_PAD = 10240
E_PAD = 327680
E_CHUNKS = E_PAD // CHUNK
E_CHUNKS_PER_W = E_CHUNKS // NW
N_CHUNKS = N_PAD // CHUNK
N_STRIPE = N_PAD // NS

BE = 640
G_VALID = N_EDGES // BE
G_TOTAL = E_PAD // BE

_f32 = jnp.float32


def _mesh():
    return plsc.VectorSubcoreMesh(
        core_axis_name="c", subcore_axis_name="s", num_cores=NC,
        num_subcores=NS)


def _wid():
    return lax.axis_index("s") * NC + lax.axis_index("c")


NBUF = 6
NB_AGG = 2


def _gather_ring(tab, idx_all, out_h, rows_v, gsems, wsems, w, nchunks,
                 idx_off=0, nbuf=NBUF, pref=4):
    base = w * nchunks * CHUNK

    def fire_gather(t):
        b = t % nbuf
        pltpu.async_copy(tab.at[idx_all.at[idx_off + t]],
                         rows_v.at[pl.ds(b * CHUNK, CHUNK)], gsems[b])

    for t in range(min(pref, nchunks)):
        fire_gather(t)
    for t in range(nchunks):
        b = t % nbuf
        pltpu.make_async_copy(tab.at[idx_all.at[idx_off + t]],
                              rows_v.at[pl.ds(b * CHUNK, CHUNK)],
                              gsems[b]).wait()
        pltpu.async_copy(rows_v.at[pl.ds(b * CHUNK, CHUNK)],
                         out_h.at[pl.ds(base + t * CHUNK, CHUNK)], wsems[b])
        c = t + pref
        if c < nchunks:
            bc = c % nbuf
            if c >= nbuf:
                pltpu.make_async_copy(
                    rows_v.at[pl.ds(bc * CHUNK, CHUNK)],
                    out_h.at[pl.ds(base + (c - nbuf) * CHUNK, CHUNK)],
                    wsems[bc]).wait()
            fire_gather(c)
    for t in range(max(0, nchunks - nbuf), nchunks):
        b = t % nbuf
        pltpu.make_async_copy(rows_v.at[pl.ds(b * CHUNK, CHUNK)],
                              out_h.at[pl.ds(base + t * CHUNK, CHUNK)],
                              wsems[b]).wait()


def _gather_scratch(nbuf=NBUF):
    return ([pltpu.VMEM((E_CHUNKS_PER_W, CHUNK), jnp.int32),
             pltpu.VMEM((nbuf * CHUNK, D), _f32)]
            + [pltpu.SemaphoreType.DMA] * (2 * nbuf))


def _sc_embed(node_tab, edge_tab, nids, eids):
    @functools.partial(
        pl.kernel,
        mesh=_mesh(),
        out_type=[
            jax.ShapeDtypeStruct((N_PAD, D), _f32),
            jax.ShapeDtypeStruct((E_PAD, D), _f32),
        ],
        scratch_types=(_gather_scratch()
                       + [pltpu.VMEM_SHARED((64, D), _f32)]),
    )
    def k(ntab, etab, nidx, eidx, x0_out, ef_out, idx_all, rows_v, *rest):
        w = _wid()
        gsems, wsems = rest[:NBUF], rest[NBUF:2 * NBUF]
        spm_e = rest[2 * NBUF]

        @pl.when(lax.axis_index("s") == 0)
        def _():
            pltpu.sync_copy(etab, spm_e)
        pltpu.sync_copy(eidx.at[pl.ds(w * E_CHUNKS_PER_W, E_CHUNKS_PER_W)],
                        idx_all)
        plsc.subcore_barrier()
        _gather_ring(spm_e, idx_all, ef_out, rows_v, gsems, wsems, w,
                     E_CHUNKS_PER_W)

        @pl.when(w < 16)
        def _():
            nper = N_CHUNKS // 16
            pltpu.sync_copy(nidx, idx_all)
            _gather_ring(ntab, idx_all, x0_out, rows_v, gsems, wsems, w,
                         nper, idx_off=w * nper)

    return k(node_tab, edge_tab, nids, eids)


def _sc_gather1(tab, idx):
    @functools.partial(
        pl.kernel,
        mesh=_mesh(),
        out_type=[jax.ShapeDtypeStruct((E_PAD, D), _f32)],
        scratch_types=(_gather_scratch(NB_AGG)
                       + [pltpu.VMEM_SHARED((N_PAD, D), _f32)]),
    )
    def k(tab_h, idx_h, out_h, idx_all, rows_v, *rest):
        gsems, wsems = rest[:NB_AGG], rest[NB_AGG:2 * NB_AGG]
        spm = rest[2 * NB_AGG]
        w = _wid()
        s = lax.axis_index("s")
        stripe = s * N_STRIPE
        pltpu.sync_copy(tab_h.at[pl.ds(stripe, N_STRIPE)],
                        spm.at[pl.ds(stripe, N_STRIPE)])
        pltpu.sync_copy(idx_h.at[pl.ds(w * E_CHUNKS_PER_W, E_CHUNKS_PER_W)],
                        idx_all)
        plsc.subcore_barrier()
        _gather_ring(spm, idx_all, out_h, rows_v, gsems, wsems, w,
                     E_CHUNKS_PER_W, nbuf=NB_AGG, pref=NB_AGG)

    return k(tab, idx)[0]


def _sc_gather2(x_tab, q_tab, src_idx, dst_idx):
    return _sc_gather1(x_tab, src_idx), _sc_gather1(q_tab, dst_idx)


def _sc_agg(msg, emul, dst_idx):
    @functools.partial(
        pl.kernel,
        mesh=_mesh(),
        out_type=[
            jax.ShapeDtypeStruct((NC * N_PAD, D), _f32),
            jax.ShapeDtypeStruct((NC * N_PAD, D), _f32),
        ],
        scratch_types=(
            [pltpu.VMEM((E_CHUNKS_PER_W, CHUNK), jnp.int32),
             pltpu.VMEM((NB_AGG * CHUNK, D), _f32),
             pltpu.VMEM_SHARED((N_PAD, D), _f32)]
            + [pltpu.SemaphoreType.DMA] * (2 * NB_AGG)),
    )
    def k(msg_h, e_h, didx, num_out, den_out, idx_all, m_v, acc, *sems):
        c = lax.axis_index("c")
        s = lax.axis_index("s")
        w = s * NC + c
        stripe = s * N_STRIPE
        z16 = jnp.zeros((16,), _f32)
        lsems, ssems = sems[:NB_AGG], sems[NB_AGG:]
        NCH = E_CHUNKS_PER_W
        base = w * NCH * CHUNK

        pltpu.sync_copy(didx.at[pl.ds(w * NCH, NCH)], idx_all)

        def zero_buf():
            def zrow(i, carry):
                r = i // (D // 16)
                kk = i % (D // 16)
                m_v[r, pl.ds(kk * 16, 16)] = z16
                return carry

            lax.fori_loop(0, CHUNK * (D // 16), zrow, 0)
            for t in range(N_STRIPE // CHUNK):
                pltpu.sync_copy(m_v.at[pl.ds(0, CHUNK)],
                                acc.at[pl.ds(stripe + t * CHUNK, CHUNK)])

        def scatter_phase(src_h, out_h):
            zero_buf()
            plsc.subcore_barrier()

            def fire_load(t):
                b = t % NB_AGG
                pltpu.async_copy(src_h.at[pl.ds(base + t * CHUNK, CHUNK)],
                                 m_v.at[pl.ds(b * CHUNK, CHUNK)], lsems[b])

            PREF = 2
            for t in range(min(PREF, NCH)):
                fire_load(t)
            for t in range(NCH):
                b = t % NB_AGG
                pltpu.make_async_copy(
                    src_h.at[pl.ds(base + t * CHUNK, CHUNK)],
                    m_v.at[pl.ds(b * CHUNK, CHUNK)], lsems[b]).wait()
                pltpu.async_copy(m_v.at[pl.ds(b * CHUNK, CHUNK)],
                                 acc.at[idx_all.at[t]], ssems[b], add=True)
                cc = t + PREF
                if cc < NCH:
                    bc = cc % NB_AGG
                    if cc >= NB_AGG:
                        pltpu.make_async_copy(
                            m_v.at[pl.ds(bc * CHUNK, CHUNK)],
                            acc.at[idx_all.at[cc - NB_AGG]],
                            ssems[bc]).wait()
                    fire_load(cc)
            for t in range(max(0, NCH - NB_AGG), NCH):
                b = t % NB_AGG
                pltpu.make_async_copy(m_v.at[pl.ds(b * CHUNK, CHUNK)],
                                      acc.at[idx_all.at[t]],
                                      ssems[b]).wait()
            plsc.subcore_barrier()
            for t in range(N_STRIPE // CHUNK):
                lo = stripe + t * CHUNK
                pltpu.sync_copy(acc.at[pl.ds(lo, CHUNK)],
                                out_h.at[pl.ds(c * N_PAD + lo, CHUNK)])

        scatter_phase(msg_h, num_out)
        scatter_phase(e_h, den_out)

    num, den = k(msg, emul, dst_idx)
    return num.reshape(NC, N_PAD, D), den.reshape(NC, N_PAD, D)


def _ln(x, s, b):
    mu = jnp.mean(x, axis=-1, keepdims=True)
    d = x - mu
    var = jnp.mean(d * d, axis=-1, keepdims=True)
    return d * lax.rsqrt(var + EPS) * s + b


def _head_selector(rows, cols):
    r = lax.broadcasted_iota(jnp.int32, (rows, cols), 0)
    ccc = lax.broadcasted_iota(jnp.int32, (rows, cols), 1)
    return (ccc // D_KV == r).astype(_f32)


def _dot(a, b):
    return jnp.dot(a, b, preferred_element_type=_f32)


def _bdot(a, b):
    return jnp.dot(a.astype(jnp.bfloat16), b.astype(jnp.bfloat16),
                   preferred_element_type=_f32)


def _tc_nodeq(x, lns, lnb, wqT):
    def body(x_ref, s_ref, b_ref, w_ref, q_ref):
        q_ref[...] = _dot(_ln(x_ref[...], s_ref[...], b_ref[...]), w_ref[...])

    return pl.pallas_call(
        body,
        out_shape=jax.ShapeDtypeStruct((N_PAD, D), _f32),
    )(x, lns.reshape(1, D), lnb.reshape(1, D), wqT)


def _tc_edge1(kv, ef, qi, a1, a2, ab, gw, gb, bw, bb, k1, k2, vw):
    def body(kv_ref, ef_ref, qi_ref, a1_ref, a2_ref, ab_ref, g_ref, gb_ref,
             bw_ref, bb_ref, k1_ref, k2_ref, v_ref,
             sc_ref, vj_ref, bm_ref):
        pid = pl.program_id(0)
        kvx = kv_ref[...]
        efx = ef_ref[...]
        z = _bdot(kvx, a1_ref[...]) + _bdot(efx, a2_ref[...]) + ab_ref[...]
        inter = jnp.where(z > 0, z, jnp.exp(jnp.minimum(z, 0.0)) - 1.0)
        gz = _bdot(inter, g_ref[...]) + gb_ref[...] + GATE_BIAS
        gate = 1.0 / (1.0 + jnp.exp(-gz))
        kadd = _bdot(inter, bw_ref[...]) + bb_ref[...]
        kvm = kvx * gate + kadd * (1.0 - gate)
        kj = _bdot(kvm, k1_ref[...]) + _bdot(efx, k2_ref[...])
        vj = _bdot(kvm, v_ref[...])
        qk = qi_ref[...] * kj
        sel = _head_selector(N_HEADS, D).T
        sc = _dot(qk, sel)
        valid = pid < G_VALID
        sc = jnp.where(valid, sc, 0.0)
        sc_ref[...] = sc
        vj_ref[...] = jnp.where(valid, vj, 0.0)
        bm_ref[...] = jnp.where(valid, jnp.max(sc, axis=0, keepdims=True),
                                -1e30).reshape(1, 1, N_HEADS)

    wspec = pl.BlockSpec((D, D), lambda i: (0, 0))
    bspec = pl.BlockSpec((1, D), lambda i: (0, 0))
    return pl.pallas_call(
        body,
        grid=(G_TOTAL,),
        in_specs=[
            pl.BlockSpec((BE, D), lambda i: (i, 0)),
            pl.BlockSpec((BE, D), lambda i: (i, 0)),
            pl.BlockSpec((BE, D), lambda i: (i, 0)),
            wspec, wspec, bspec, wspec, bspec, wspec, bspec, wspec, wspec,
            wspec,
        ],
        out_specs=[
            pl.BlockSpec((BE, N_HEADS), lambda i: (i, 0)),
            pl.BlockSpec((BE, D), lambda i: (i, 0)),
            pl.BlockSpec((1, 1, N_HEADS), lambda i: (i, 0, 0)),
        ],
        out_shape=[
            jax.ShapeDtypeStruct((E_PAD, N_HEADS), _f32),
            jax.ShapeDtypeStruct((E_PAD, D), _f32),
            jax.ShapeDtypeStruct((G_TOTAL, 1, N_HEADS), _f32),
        ],
    )(kv, ef, qi, a1, a2, ab, gw, gb, bw, bb, k1, k2, vw)


def _tc_edge2(scores, vj, bmax):
    def body(sc_ref, vj_ref, bm_ref, emul_ref, msg_ref):
        c = jnp.max(bm_ref[...])
        e = jnp.exp(sc_ref[...] - c)
        e = jnp.where(pl.program_id(0) < G_VALID, e, 0.0)
        sel = _head_selector(N_HEADS, D)
        emul = _dot(e, sel)
        msg_ref[...] = vj_ref[...] * emul
        emul_ref[...] = emul

    return pl.pallas_call(
        body,
        grid=(G_TOTAL,),
        in_specs=[
            pl.BlockSpec((BE, N_HEADS), lambda i: (i, 0)),
            pl.BlockSpec((BE, D), lambda i: (i, 0)),
            pl.BlockSpec((G_TOTAL, 1, N_HEADS), lambda i: (0, 0, 0)),
        ],
        out_specs=[
            pl.BlockSpec((BE, D), lambda i: (i, 0)),
            pl.BlockSpec((BE, D), lambda i: (i, 0)),
        ],
        out_shape=[
            jax.ShapeDtypeStruct((E_PAD, D), _f32),
            jax.ShapeDtypeStruct((E_PAD, D), _f32),
        ],
    )(scores, vj, bmax)


def _tc_nodeo(x, nums, dens, woT, ffs, ffb, wiT, wib, woT2, wob):
    def body(x_ref, num_ref, den_ref, wo_ref, ffs_ref, ffb_ref, wi_ref,
             wib_ref, wo2_ref, wob_ref, out_ref):
        num = num_ref[0] + num_ref[1]
        den = den_ref[0] + den_ref[1]
        agg = num / (den + 1e-30)
        x1 = x_ref[...] + _dot(agg, wo_ref[...])
        y = _ln(x1, ffs_ref[...], ffb_ref[...])
        h = jnp.maximum(_dot(y, wi_ref[...]) + wib_ref[...], 0.0)
        out_ref[...] = x1 + _dot(h, wo2_ref[...]) + wob_ref[...]

    return pl.pallas_call(
        body,
        out_shape=jax.ShapeDtypeStruct((N_PAD, D), _f32),
    )(x, nums, dens, woT, ffs.reshape(1, D), ffb.reshape(1, D), wiT,
      wib.reshape(1, D), woT2, wob.reshape(1, D))


def kernel(node_ids, edge_index, edge_ids, node_emb, edge_emb, ln_scale,
           ln_bias, Wq, relA_w, relA_b, relG_w, relG_b, relB_w, relB_b, Wk,
           Wv, Wo, ff_ln_scale, ff_ln_bias, wi_w, wi_b, wo_w, wo_b):
    nids = jnp.pad(node_ids.astype(jnp.int32), (0, N_PAD - N_NODES))
    nids = nids.reshape(N_CHUNKS, CHUNK)
    src = jnp.pad(edge_index[0].astype(jnp.int32), (0, E_PAD - N_EDGES))
    dst = jnp.pad(edge_index[1].astype(jnp.int32), (0, E_PAD - N_EDGES))
    src = src.reshape(E_CHUNKS, CHUNK)
    dst = dst.reshape(E_CHUNKS, CHUNK)
    eids = jnp.pad(edge_ids.astype(jnp.int32), (0, E_PAD - N_EDGES))
    eids = eids.reshape(E_CHUNKS, CHUNK)

    x, ef = _sc_embed(node_emb, edge_emb, nids, eids)

    for l in range(2):
        q = _tc_nodeq(x, ln_scale[l], ln_bias[l], Wq[l].T)
        kv, qi = _sc_gather2(x, q, src, dst)
        scores, vj, bmax = _tc_edge1(
            kv, ef, qi,
            relA_w[l, :, :D].T, relA_w[l, :, D:].T, relA_b[l].reshape(1, D),
            relG_w[l].T, relG_b[l].reshape(1, D),
            relB_w[l].T, relB_b[l].reshape(1, D),
            Wk[l, :, :D].T, Wk[l, :, D:].T, Wv[l].T)
        emul, msg = _tc_edge2(scores, vj, bmax)
        nums, dens = _sc_agg(msg, emul, dst)
        x = _tc_nodeo(x, nums, dens, Wo[l].T, ff_ln_scale[l], ff_ln_bias[l],
                      wi_w[l].T, wi_b[l], wo_w[l].T, wo_b[l])

    return x[:N_NODES]


# speedup vs baseline: 1.0049x; 1.0049x over previous
import functools

import jax
import jax.numpy as jnp
from jax import lax
from jax.experimental import pallas as pl
from jax.experimental.pallas import tpu as pltpu
from jax.experimental.pallas import tpu_sc as plsc

N_NODES = 10000
N_EDGES = 320000
D = 128
N_HEADS = 8
D_KV = 16
EPS = 1e-6
GATE_BIAS = 3.0

NC = 2
NS = 16
NW = NC * NS

CHUNK = 128
N_PAD = 10240
E_PAD = 327680
E_CHUNKS = E_PAD // CHUNK
E_CHUNKS_PER_W = E_CHUNKS // NW
N_CHUNKS = N_PAD // CHUNK
N_STRIPE = N_PAD // NS

BE = 640
G_VALID = N_EDGES // BE
G_TOTAL = E_PAD // BE

_f32 = jnp.float32


def _mesh():
    return plsc.VectorSubcoreMesh(
        core_axis_name="c", subcore_axis_name="s", num_cores=NC,
        num_subcores=NS)


def _wid():
    return lax.axis_index("s") * NC + lax.axis_index("c")


NBUF = 6
NB_AGG = 2


def _gather_ring(tab, idx_all, out_h, rows_v, gsems, wsems, w, nchunks,
                 idx_off=0, nbuf=NBUF, pref=4):
    base = w * nchunks * CHUNK

    def fire_gather(t):
        b = t % nbuf
        pltpu.async_copy(tab.at[idx_all.at[idx_off + t]],
                         rows_v.at[pl.ds(b * CHUNK, CHUNK)], gsems[b])

    for t in range(min(pref, nchunks)):
        fire_gather(t)
    for t in range(nchunks):
        b = t % nbuf
        pltpu.make_async_copy(tab.at[idx_all.at[idx_off + t]],
                              rows_v.at[pl.ds(b * CHUNK, CHUNK)],
                              gsems[b]).wait()
        pltpu.async_copy(rows_v.at[pl.ds(b * CHUNK, CHUNK)],
                         out_h.at[pl.ds(base + t * CHUNK, CHUNK)], wsems[b])
        c = t + pref
        if c < nchunks:
            bc = c % nbuf
            if c >= nbuf:
                pltpu.make_async_copy(
                    rows_v.at[pl.ds(bc * CHUNK, CHUNK)],
                    out_h.at[pl.ds(base + (c - nbuf) * CHUNK, CHUNK)],
                    wsems[bc]).wait()
            fire_gather(c)
    for t in range(max(0, nchunks - nbuf), nchunks):
        b = t % nbuf
        pltpu.make_async_copy(rows_v.at[pl.ds(b * CHUNK, CHUNK)],
                              out_h.at[pl.ds(base + t * CHUNK, CHUNK)],
                              wsems[b]).wait()


def _gather_scratch(nbuf=NBUF):
    return ([pltpu.VMEM((E_CHUNKS_PER_W, CHUNK), jnp.int32),
             pltpu.VMEM((nbuf * CHUNK, D), _f32)]
            + [pltpu.SemaphoreType.DMA] * (2 * nbuf))


def _sc_embed(node_tab, edge_tab, nids, eids):
    @functools.partial(
        pl.kernel,
        mesh=_mesh(),
        out_type=[
            jax.ShapeDtypeStruct((N_PAD, D), _f32),
            jax.ShapeDtypeStruct((E_PAD, D), _f32),
        ],
        scratch_types=(_gather_scratch()
                       + [pltpu.VMEM_SHARED((64, D), _f32)]),
    )
    def k(ntab, etab, nidx, eidx, x0_out, ef_out, idx_all, rows_v, *rest):
        w = _wid()
        gsems, wsems = rest[:NBUF], rest[NBUF:2 * NBUF]
        spm_e = rest[2 * NBUF]

        @pl.when(lax.axis_index("s") == 0)
        def _():
            pltpu.sync_copy(etab, spm_e)
        pltpu.sync_copy(eidx.at[pl.ds(w * E_CHUNKS_PER_W, E_CHUNKS_PER_W)],
                        idx_all)
        plsc.subcore_barrier()
        _gather_ring(spm_e, idx_all, ef_out, rows_v, gsems, wsems, w,
                     E_CHUNKS_PER_W)

        @pl.when(w < 16)
        def _():
            nper = N_CHUNKS // 16
            pltpu.sync_copy(nidx, idx_all)
            _gather_ring(ntab, idx_all, x0_out, rows_v, gsems, wsems, w,
                         nper, idx_off=w * nper)

    return k(node_tab, edge_tab, nids, eids)


def _sc_gather1(tab, idx):
    @functools.partial(
        pl.kernel,
        mesh=_mesh(),
        out_type=[jax.ShapeDtypeStruct((E_PAD, D), _f32)],
        scratch_types=(_gather_scratch(NB_AGG)
                       + [pltpu.VMEM_SHARED((N_PAD, D), _f32)]),
    )
    def k(tab_h, idx_h, out_h, idx_all, rows_v, *rest):
        gsems, wsems = rest[:NB_AGG], rest[NB_AGG:2 * NB_AGG]
        spm = rest[2 * NB_AGG]
        w = _wid()
        s = lax.axis_index("s")
        stripe = s * N_STRIPE
        pltpu.sync_copy(tab_h.at[pl.ds(stripe, N_STRIPE)],
                        spm.at[pl.ds(stripe, N_STRIPE)])
        pltpu.sync_copy(idx_h.at[pl.ds(w * E_CHUNKS_PER_W, E_CHUNKS_PER_W)],
                        idx_all)
        plsc.subcore_barrier()
        _gather_ring(spm, idx_all, out_h, rows_v, gsems, wsems, w,
                     E_CHUNKS_PER_W, nbuf=NB_AGG, pref=NB_AGG)

    return k(tab, idx)[0]


def _sc_gather2(x_tab, q_tab, src_idx, dst_idx):
    return _sc_gather1(x_tab, src_idx), _sc_gather1(q_tab, dst_idx)


def _sc_agg(msg, emul, dst_idx):
    @functools.partial(
        pl.kernel,
        mesh=_mesh(),
        out_type=[
            jax.ShapeDtypeStruct((NC * N_PAD, D), _f32),
            jax.ShapeDtypeStruct((NC * N_PAD, D), _f32),
        ],
        scratch_types=(
            [pltpu.VMEM((E_CHUNKS_PER_W, CHUNK), jnp.int32),
             pltpu.VMEM((NB_AGG * CHUNK, D), _f32),
             pltpu.VMEM_SHARED((N_PAD, D), _f32)]
            + [pltpu.SemaphoreType.DMA] * (2 * NB_AGG)),
    )
    def k(msg_h, e_h, didx, num_out, den_out, idx_all, m_v, acc, *sems):
        c = lax.axis_index("c")
        s = lax.axis_index("s")
        w = s * NC + c
        stripe = s * N_STRIPE
        z16 = jnp.zeros((16,), _f32)
        lsems, ssems = sems[:NB_AGG], sems[NB_AGG:]
        NCH = E_CHUNKS_PER_W
        base = w * NCH * CHUNK

        pltpu.sync_copy(didx.at[pl.ds(w * NCH, NCH)], idx_all)

        def zero_buf():
            def zrow(i, carry):
                r = i // (D // 16)
                kk = i % (D // 16)
                m_v[r, pl.ds(kk * 16, 16)] = z16
                return carry

            lax.fori_loop(0, CHUNK * (D // 16), zrow, 0)
            for t in range(N_STRIPE // CHUNK):
                pltpu.sync_copy(m_v.at[pl.ds(0, CHUNK)],
                                acc.at[pl.ds(stripe + t * CHUNK, CHUNK)])

        def scatter_phase(src_h, out_h):
            zero_buf()
            plsc.subcore_barrier()

            def fire_load(t):
                b = t % NB_AGG
                pltpu.async_copy(src_h.at[pl.ds(base + t * CHUNK, CHUNK)],
                                 m_v.at[pl.ds(b * CHUNK, CHUNK)], lsems[b])

            PREF = 2
            for t in range(min(PREF, NCH)):
                fire_load(t)
            for t in range(NCH):
                b = t % NB_AGG
                pltpu.make_async_copy(
                    src_h.at[pl.ds(base + t * CHUNK, CHUNK)],
                    m_v.at[pl.ds(b * CHUNK, CHUNK)], lsems[b]).wait()
                pltpu.async_copy(m_v.at[pl.ds(b * CHUNK, CHUNK)],
                                 acc.at[idx_all.at[t]], ssems[b], add=True)
                cc = t + PREF
                if cc < NCH:
                    bc = cc % NB_AGG
                    if cc >= NB_AGG:
                        pltpu.make_async_copy(
                            m_v.at[pl.ds(bc * CHUNK, CHUNK)],
                            acc.at[idx_all.at[cc - NB_AGG]],
                            ssems[bc]).wait()
                    fire_load(cc)
            for t in range(max(0, NCH - NB_AGG), NCH):
                b = t % NB_AGG
                pltpu.make_async_copy(m_v.at[pl.ds(b * CHUNK, CHUNK)],
                                      acc.at[idx_all.at[t]],
                                      ssems[b]).wait()
            plsc.subcore_barrier()
            for t in range(N_STRIPE // CHUNK):
                lo = stripe + t * CHUNK
                pltpu.sync_copy(acc.at[pl.ds(lo, CHUNK)],
                                out_h.at[pl.ds(c * N_PAD + lo, CHUNK)])

        scatter_phase(msg_h, num_out)
        scatter_phase(e_h, den_out)

    num, den = k(msg, emul, dst_idx)
    return num.reshape(NC, N_PAD, D), den.reshape(NC, N_PAD, D)


def _ln(x, s, b):
    mu = jnp.mean(x, axis=-1, keepdims=True)
    d = x - mu
    var = jnp.mean(d * d, axis=-1, keepdims=True)
    return d * lax.rsqrt(var + EPS) * s + b


def _head_selector(rows, cols):
    r = lax.broadcasted_iota(jnp.int32, (rows, cols), 0)
    ccc = lax.broadcasted_iota(jnp.int32, (rows, cols), 1)
    return (ccc // D_KV == r).astype(_f32)


def _dot(a, b):
    return jnp.dot(a, b, preferred_element_type=_f32)


def _tc_nodeq(x, lns, lnb, wqT):
    def body(x_ref, s_ref, b_ref, w_ref, q_ref):
        q_ref[...] = _dot(_ln(x_ref[...], s_ref[...], b_ref[...]), w_ref[...])

    return pl.pallas_call(
        body,
        out_shape=jax.ShapeDtypeStruct((N_PAD, D), _f32),
    )(x, lns.reshape(1, D), lnb.reshape(1, D), wqT)


def _tc_edge1(kv, ef, qi, a1, a2, ab, gw, gb, bw, bb, k1, k2, vw):
    def body(kv_ref, ef_ref, qi_ref, a1_ref, a2_ref, ab_ref, g_ref, gb_ref,
             bw_ref, bb_ref, k1_ref, k2_ref, v_ref,
             sc_ref, vj_ref, bm_ref):
        pid = pl.program_id(0)
        kvx = kv_ref[...]
        efx = ef_ref[...]
        z = _dot(kvx, a1_ref[...]) + _dot(efx, a2_ref[...]) + ab_ref[...]
        inter = jnp.where(z > 0, z, jnp.exp(jnp.minimum(z, 0.0)) - 1.0)
        gz = _dot(inter, g_ref[...]) + gb_ref[...] + GATE_BIAS
        gate = 1.0 / (1.0 + jnp.exp(-gz))
        kadd = _dot(inter, bw_ref[...]) + bb_ref[...]
        kvm = kvx * gate + kadd * (1.0 - gate)
        kj = _dot(kvm, k1_ref[...]) + _dot(efx, k2_ref[...])
        vj = _dot(kvm, v_ref[...])
        qk = qi_ref[...] * kj
        sel = _head_selector(N_HEADS, D).T
        sc = _dot(qk, sel)
        valid = pid < G_VALID
        sc = jnp.where(valid, sc, 0.0)
        sc_ref[...] = sc
        vj_ref[...] = jnp.where(valid, vj, 0.0)
        bm_ref[...] = jnp.where(valid, jnp.max(sc, axis=0, keepdims=True),
                                -1e30).reshape(1, 1, N_HEADS)

    wspec = pl.BlockSpec((D, D), lambda i: (0, 0))
    bspec = pl.BlockSpec((1, D), lambda i: (0, 0))
    return pl.pallas_call(
        body,
        grid=(G_TOTAL,),
        in_specs=[
            pl.BlockSpec((BE, D), lambda i: (i, 0)),
            pl.BlockSpec((BE, D), lambda i: (i, 0)),
            pl.BlockSpec((BE, D), lambda i: (i, 0)),
            wspec, wspec, bspec, wspec, bspec, wspec, bspec, wspec, wspec,
            wspec,
        ],
        out_specs=[
            pl.BlockSpec((BE, N_HEADS), lambda i: (i, 0)),
            pl.BlockSpec((BE, D), lambda i: (i, 0)),
            pl.BlockSpec((1, 1, N_HEADS), lambda i: (i, 0, 0)),
        ],
        out_shape=[
            jax.ShapeDtypeStruct((E_PAD, N_HEADS), _f32),
            jax.ShapeDtypeStruct((E_PAD, D), _f32),
            jax.ShapeDtypeStruct((G_TOTAL, 1, N_HEADS), _f32),
        ],
    )(kv, ef, qi, a1, a2, ab, gw, gb, bw, bb, k1, k2, vw)


def _tc_edge2(scores, vj, bmax):
    def body(sc_ref, vj_ref, bm_ref, emul_ref, msg_ref):
        c = jnp.max(bm_ref[...])
        e = jnp.exp(sc_ref[...] - c)
        e = jnp.where(pl.program_id(0) < G_VALID, e, 0.0)
        sel = _head_selector(N_HEADS, D)
        emul = _dot(e, sel)
        msg_ref[...] = vj_ref[...] * emul
        emul_ref[...] = emul

    return pl.pallas_call(
        body,
        grid=(G_TOTAL,),
        in_specs=[
            pl.BlockSpec((BE, N_HEADS), lambda i: (i, 0)),
            pl.BlockSpec((BE, D), lambda i: (i, 0)),
            pl.BlockSpec((G_TOTAL, 1, N_HEADS), lambda i: (0, 0, 0)),
        ],
        out_specs=[
            pl.BlockSpec((BE, D), lambda i: (i, 0)),
            pl.BlockSpec((BE, D), lambda i: (i, 0)),
        ],
        out_shape=[
            jax.ShapeDtypeStruct((E_PAD, D), _f32),
            jax.ShapeDtypeStruct((E_PAD, D), _f32),
        ],
    )(scores, vj, bmax)


def _tc_nodeo(x, nums, dens, woT, ffs, ffb, wiT, wib, woT2, wob):
    def body(x_ref, num_ref, den_ref, wo_ref, ffs_ref, ffb_ref, wi_ref,
             wib_ref, wo2_ref, wob_ref, out_ref):
        num = num_ref[0] + num_ref[1]
        den = den_ref[0] + den_ref[1]
        agg = num / (den + 1e-30)
        x1 = x_ref[...] + _dot(agg, wo_ref[...])
        y = _ln(x1, ffs_ref[...], ffb_ref[...])
        h = jnp.maximum(_dot(y, wi_ref[...]) + wib_ref[...], 0.0)
        out_ref[...] = x1 + _dot(h, wo2_ref[...]) + wob_ref[...]

    return pl.pallas_call(
        body,
        out_shape=jax.ShapeDtypeStruct((N_PAD, D), _f32),
    )(x, nums, dens, woT, ffs.reshape(1, D), ffb.reshape(1, D), wiT,
      wib.reshape(1, D), woT2, wob.reshape(1, D))


def kernel(node_ids, edge_index, edge_ids, node_emb, edge_emb, ln_scale,
           ln_bias, Wq, relA_w, relA_b, relG_w, relG_b, relB_w, relB_b, Wk,
           Wv, Wo, ff_ln_scale, ff_ln_bias, wi_w, wi_b, wo_w, wo_b):
    nids = jnp.pad(node_ids.astype(jnp.int32), (0, N_PAD - N_NODES))
    nids = nids.reshape(N_CHUNKS, CHUNK)
    src = jnp.pad(edge_index[0].astype(jnp.int32), (0, E_PAD - N_EDGES))
    dst = jnp.pad(edge_index[1].astype(jnp.int32), (0, E_PAD - N_EDGES))
    src = src.reshape(E_CHUNKS, CHUNK)
    dst = dst.reshape(E_CHUNKS, CHUNK)
    eids = jnp.pad(edge_ids.astype(jnp.int32), (0, E_PAD - N_EDGES))
    eids = eids.reshape(E_CHUNKS, CHUNK)

    x, ef = _sc_embed(node_emb, edge_emb, nids, eids)

    for l in range(2):
        q = _tc_nodeq(x, ln_scale[l], ln_bias[l], Wq[l].T)
        kv, qi = _sc_gather2(x, q, src, dst)
        scores, vj, bmax = _tc_edge1(
            kv, ef, qi,
            relA_w[l, :, :D].T, relA_w[l, :, D:].T, relA_b[l].reshape(1, D),
            relG_w[l].T, relG_b[l].reshape(1, D),
            relB_w[l].T, relB_b[l].reshape(1, D),
            Wk[l, :, :D].T, Wk[l, :, D:].T, Wv[l].T)
        emul, msg = _tc_edge2(scores, vj, bmax)
        nums, dens = _sc_agg(msg, emul, dst)
        x = _tc_nodeo(x, nums, dens, Wo[l].T, ff_ln_scale[l], ff_ln_bias[l],
                      wi_w[l].T, wi_b[l], wo_w[l].T, wo_b[l])

    return x[:N_NODES]
